# baseline (speedup 1.0000x reference)
import math

import jax
import jax.numpy as jnp
import numpy as np
from jax.experimental import pallas as pl
from jax.experimental.pallas import tpu as pltpu

B = 4
S = 1024
H = 16
KVH = 4
G = H // KVH
D = 128
WINDOW = 512
SCALE = 0.08838834764831845

BQ = 256
KS = BQ + WINDOW
BC = 256
NC = KS // BC
NQ = S // BQ


def _slopes(n):
    def pow2(n):
        start = 2 ** (-(2 ** (-(math.log2(n) - 3))))
        return [start * start ** i for i in range(n)]
    if math.log2(n).is_integer():
        return pow2(n)
    closest = 2 ** math.floor(math.log2(n))
    return pow2(closest) + _slopes(2 * closest)[0::2][: n - closest]


def _attn_kernel(slopes_ref, q_ref, k_ref, v_ref, o_ref):
    h = pl.program_id(1)
    qi = pl.program_id(2)
    q_base = qi * BQ
    start = pl.multiple_of(jnp.maximum(q_base + BQ - KS, 0), BQ)

    kspan = k_ref[0, pl.ds(start, KS), :]
    vspan = v_ref[0, pl.ds(start, KS), :]
    i_idx = q_base + jax.lax.broadcasted_iota(jnp.int32, (BQ, KS), 0)
    j_idx = start + jax.lax.broadcasted_iota(jnp.int32, (BQ, KS), 1)
    valid = (j_idx <= i_idx) & (j_idx >= i_idx - WINDOW)
    delta_masked = jnp.where(
        valid, (j_idx - i_idx).astype(jnp.float32), jnp.float32(-1e30))

    for g in range(G):
        qg = q_ref[0, :, g * D:(g + 1) * D] * jnp.float32(SCALE)
        s = jax.lax.dot_general(
            qg, kspan, (((1,), (1,)), ((), ())),
            preferred_element_type=jnp.float32,
        )
        p = jnp.exp(s + slopes_ref[h, g] * delta_masked)
        l = jnp.sum(p, axis=1, keepdims=True)
        og = jax.lax.dot_general(
            p, vspan, (((1,), (0,)), ((), ())),
            preferred_element_type=jnp.float32,
        )
        o_ref[0, :, g * D:(g + 1) * D] = og * (1.0 / l)


def kernel(q, k, v):
    qh = q.reshape(B, S, H * D)
    kh = k.reshape(B, S, KVH * D)
    vh = v.reshape(B, S, KVH * D)
    slopes = jnp.asarray(
        np.array(_slopes(H), dtype=np.float32).reshape(KVH, G))

    out = pl.pallas_call(
        _attn_kernel,
        grid=(B, KVH, NQ),
        in_specs=[
            pl.BlockSpec(memory_space=pltpu.SMEM),
            pl.BlockSpec((1, BQ, G * D), lambda b, h, qi: (b, qi, h)),
            pl.BlockSpec((1, S, D), lambda b, h, qi: (b, 0, h)),
            pl.BlockSpec((1, S, D), lambda b, h, qi: (b, 0, h)),
        ],
        out_specs=pl.BlockSpec((1, BQ, G * D), lambda b, h, qi: (b, qi, h)),
        out_shape=jax.ShapeDtypeStruct((B, S, H * D), jnp.float32),
        compiler_params=pltpu.CompilerParams(
            dimension_semantics=("parallel", "parallel", "arbitrary")),
    )(slopes, qh, kh, vh)
    return out.reshape(B * S, H * D)


# speedup vs baseline: 1.6914x; 1.6914x over previous
import math

import jax
import jax.numpy as jnp
import numpy as np
from jax.experimental import pallas as pl
from jax.experimental.pallas import tpu as pltpu

B = 4
S = 1024
H = 16
KVH = 4
G = H // KVH
D = 128
WINDOW = 512
SCALE = 0.08838834764831845

BQ = 256
NQ = S // BQ


def _slopes(n):
    def pow2(n):
        start = 2 ** (-(2 ** (-(math.log2(n) - 3))))
        return [start * start ** i for i in range(n)]
    if math.log2(n).is_integer():
        return pow2(n)
    closest = 2 ** math.floor(math.log2(n))
    return pow2(closest) + _slopes(2 * closest)[0::2][: n - closest]


def _attn_kernel(slopes_ref, q_ref, k_ref, v_ref, o_ref):
    h = pl.program_id(1)

    for qi in range(NQ):
        q_base = qi * BQ
        start = max(q_base + BQ - (BQ + WINDOW), 0)
        ks = q_base + BQ - start

        kspan = k_ref[0, start:start + ks, :]
        vspan = v_ref[0, start:start + ks, :]
        i_idx = q_base + jax.lax.broadcasted_iota(jnp.int32, (BQ, ks), 0)
        j_idx = start + jax.lax.broadcasted_iota(jnp.int32, (BQ, ks), 1)
        valid = (j_idx <= i_idx) & (j_idx >= i_idx - WINDOW)
        delta_masked = jnp.where(
            valid, (j_idx - i_idx).astype(jnp.float32), jnp.float32(-1e30))

        for g in range(G):
            qg = q_ref[0, q_base:q_base + BQ, g * D:(g + 1) * D] \
                * jnp.float32(SCALE)
            s = jax.lax.dot_general(
                qg, kspan, (((1,), (1,)), ((), ())),
                preferred_element_type=jnp.float32,
            )
            p = jnp.exp(s + slopes_ref[h, g] * delta_masked)
            l = jnp.sum(p, axis=1, keepdims=True)
            og = jax.lax.dot_general(
                p, vspan, (((1,), (0,)), ((), ())),
                preferred_element_type=jnp.float32,
            )
            o_ref[0, q_base:q_base + BQ, g * D:(g + 1) * D] = og * (1.0 / l)


def kernel(q, k, v):
    qh = q.reshape(B, S, H * D)
    kh = k.reshape(B, S, KVH * D)
    vh = v.reshape(B, S, KVH * D)
    slopes = jnp.asarray(
        np.array(_slopes(H), dtype=np.float32).reshape(KVH, G))

    out = pl.pallas_call(
        _attn_kernel,
        grid=(B, KVH),
        in_specs=[
            pl.BlockSpec(memory_space=pltpu.SMEM),
            pl.BlockSpec((1, S, G * D), lambda b, h: (b, 0, h)),
            pl.BlockSpec((1, S, D), lambda b, h: (b, 0, h)),
            pl.BlockSpec((1, S, D), lambda b, h: (b, 0, h)),
        ],
        out_specs=pl.BlockSpec((1, S, G * D), lambda b, h: (b, 0, h)),
        out_shape=jax.ShapeDtypeStruct((B, S, H * D), jnp.float32),
        compiler_params=pltpu.CompilerParams(
            dimension_semantics=("parallel", "parallel")),
    )(slopes, qh, kh, vh)
    return out.reshape(B * S, H * D)


# speedup vs baseline: 1.7068x; 1.0091x over previous
import math

import jax
import jax.numpy as jnp
import numpy as np
from jax.experimental import pallas as pl
from jax.experimental.pallas import tpu as pltpu

B = 4
S = 1024
H = 16
KVH = 4
G = H // KVH
D = 128
WINDOW = 512
SCALE = 0.08838834764831845
LOG2E = 1.4426950408889634

BQ = 256
NQ = S // BQ


def _slopes(n):
    def pow2(n):
        start = 2 ** (-(2 ** (-(math.log2(n) - 3))))
        return [start * start ** i for i in range(n)]
    if math.log2(n).is_integer():
        return pow2(n)
    closest = 2 ** math.floor(math.log2(n))
    return pow2(closest) + _slopes(2 * closest)[0::2][: n - closest]


def _attn_kernel(slopes_ref, q_ref, k_ref, v_ref, o_ref):
    h = pl.program_id(1)

    for qi in range(NQ):
        q_base = qi * BQ
        start = max(q_base + BQ - (BQ + WINDOW), 0)
        ks = q_base + BQ - start

        kspan = k_ref[0, start:start + ks, :]
        vspan = v_ref[0, start:start + ks, :]
        i_idx = q_base + jax.lax.broadcasted_iota(jnp.int32, (BQ, ks), 0)
        j_idx = start + jax.lax.broadcasted_iota(jnp.int32, (BQ, ks), 1)
        valid = (j_idx <= i_idx) & (j_idx >= i_idx - WINDOW)
        delta_masked = jnp.where(
            valid, (j_idx - i_idx).astype(jnp.float32), jnp.float32(-1e30))

        for g in range(G):
            qg = q_ref[0, q_base:q_base + BQ, g * D:(g + 1) * D] \
                * jnp.float32(SCALE * LOG2E)
            s = jax.lax.dot_general(
                qg, kspan, (((1,), (1,)), ((), ())),
                preferred_element_type=jnp.float32,
            )
            p = jnp.exp2(s + slopes_ref[h, g] * delta_masked)
            l = jnp.sum(p, axis=1, keepdims=True)
            og = jax.lax.dot_general(
                p, vspan, (((1,), (0,)), ((), ())),
                preferred_element_type=jnp.float32,
            )
            o_ref[0, q_base:q_base + BQ, g * D:(g + 1) * D] = og * (1.0 / l)


def kernel(q, k, v):
    qh = q.reshape(B, S, H * D)
    kh = k.reshape(B, S, KVH * D)
    vh = v.reshape(B, S, KVH * D)
    slopes = jnp.asarray(
        (np.array(_slopes(H), dtype=np.float64) * LOG2E)
        .astype(np.float32).reshape(KVH, G))

    out = pl.pallas_call(
        _attn_kernel,
        grid=(B, KVH),
        in_specs=[
            pl.BlockSpec(memory_space=pltpu.SMEM),
            pl.BlockSpec((1, S, G * D), lambda b, h: (b, 0, h)),
            pl.BlockSpec((1, S, D), lambda b, h: (b, 0, h)),
            pl.BlockSpec((1, S, D), lambda b, h: (b, 0, h)),
        ],
        out_specs=pl.BlockSpec((1, S, G * D), lambda b, h: (b, 0, h)),
        out_shape=jax.ShapeDtypeStruct((B, S, H * D), jnp.float32),
        compiler_params=pltpu.CompilerParams(
            dimension_semantics=("parallel", "parallel")),
    )(slopes, qh, kh, vh)
    return out.reshape(B * S, H * D)


# speedup vs baseline: 1.7092x; 1.0014x over previous
import math

import jax
import jax.numpy as jnp
import numpy as np
from jax.experimental import pallas as pl
from jax.experimental.pallas import tpu as pltpu

B = 4
S = 1024
H = 16
KVH = 4
G = H // KVH
D = 128
WINDOW = 512
SCALE = 0.08838834764831845
LOG2E = 1.4426950408889634

BQ = 256
NQ = S // BQ


def _slopes(n):
    def pow2(n):
        start = 2 ** (-(2 ** (-(math.log2(n) - 3))))
        return [start * start ** i for i in range(n)]
    if math.log2(n).is_integer():
        return pow2(n)
    closest = 2 ** math.floor(math.log2(n))
    return pow2(closest) + _slopes(2 * closest)[0::2][: n - closest]


BLOCKS = (
    (0, 256, 0, 256),
    (256, 256, 0, 512),
    (512, 256, 0, 768),
    (768, 256, 256, 768),
)


def _attn_kernel(slopes_ref, q_ref, k_ref, v_ref, o_ref):
    h = pl.program_id(1)

    k_scaled = k_ref[0, :, :] * jnp.float32(SCALE * LOG2E)

    for q_base, bq, start, ks in BLOCKS:
        kspan = k_scaled[start:start + ks, :]
        vspan = v_ref[0, start:start + ks, :]
        i_idx = q_base + jax.lax.broadcasted_iota(jnp.int32, (bq, ks), 0)
        j_idx = start + jax.lax.broadcasted_iota(jnp.int32, (bq, ks), 1)
        valid = (j_idx <= i_idx) & (j_idx >= i_idx - WINDOW)
        delta_masked = jnp.where(
            valid, (j_idx - i_idx).astype(jnp.float32), jnp.float32(-1e30))

        for g in range(G):
            qg = q_ref[0, q_base:q_base + bq, g * D:(g + 1) * D]
            s = jax.lax.dot_general(
                qg, kspan, (((1,), (1,)), ((), ())),
                preferred_element_type=jnp.float32,
            )
            p = jnp.exp2(s + slopes_ref[h, g] * delta_masked)
            l = jnp.sum(p, axis=1, keepdims=True)
            og = jax.lax.dot_general(
                p, vspan, (((1,), (0,)), ((), ())),
                preferred_element_type=jnp.float32,
            )
            o_ref[0, q_base:q_base + bq, g * D:(g + 1) * D] = og * (1.0 / l)


def kernel(q, k, v):
    qh = q.reshape(B, S, H * D)
    kh = k.reshape(B, S, KVH * D)
    vh = v.reshape(B, S, KVH * D)
    slopes = jnp.asarray(
        (np.array(_slopes(H), dtype=np.float64) * LOG2E)
        .astype(np.float32).reshape(KVH, G))

    out = pl.pallas_call(
        _attn_kernel,
        grid=(B, KVH),
        in_specs=[
            pl.BlockSpec(memory_space=pltpu.SMEM),
            pl.BlockSpec((1, S, G * D), lambda b, h: (b, 0, h)),
            pl.BlockSpec((1, S, D), lambda b, h: (b, 0, h)),
            pl.BlockSpec((1, S, D), lambda b, h: (b, 0, h)),
        ],
        out_specs=pl.BlockSpec((1, S, G * D), lambda b, h: (b, 0, h)),
        out_shape=jax.ShapeDtypeStruct((B, S, H * D), jnp.float32),
        compiler_params=pltpu.CompilerParams(
            dimension_semantics=("parallel", "parallel")),
    )(slopes, qh, kh, vh)
    return out.reshape(B * S, H * D)


# speedup vs baseline: 1.7604x; 1.0299x over previous
import math

import jax
import jax.numpy as jnp
import numpy as np
from jax.experimental import pallas as pl
from jax.experimental.pallas import tpu as pltpu

B = 4
S = 1024
H = 16
KVH = 4
G = H // KVH
D = 128
WINDOW = 512
SCALE = 0.08838834764831845
LOG2E = 1.4426950408889634

BQ = 256
NQ = S // BQ


def _slopes(n):
    def pow2(n):
        start = 2 ** (-(2 ** (-(math.log2(n) - 3))))
        return [start * start ** i for i in range(n)]
    if math.log2(n).is_integer():
        return pow2(n)
    closest = 2 ** math.floor(math.log2(n))
    return pow2(closest) + _slopes(2 * closest)[0::2][: n - closest]


BLOCKS = (
    (0, 256, 0, 256),
    (256, 256, 0, 512),
    (512, 256, 0, 768),
    (768, 256, 256, 768),
)


def _attn_kernel(slopes_ref, q_ref, k_ref, v_ref, o_ref):
    h = pl.program_id(1)

    k_scaled = k_ref[0, :, :] * jnp.float32(SCALE * LOG2E)

    for q_base, bq, start, ks in BLOCKS:
        kspan = k_scaled[start:start + ks, :]
        vspan = v_ref[0, start:start + ks, :]
        i_idx = q_base + jax.lax.broadcasted_iota(jnp.int32, (bq, ks), 0)
        j_idx = start + jax.lax.broadcasted_iota(jnp.int32, (bq, ks), 1)
        valid = (j_idx <= i_idx) & (j_idx >= i_idx - WINDOW)
        delta_masked = jnp.where(
            valid, (j_idx - i_idx).astype(jnp.float32), jnp.float32(-1e30))

        qall = jnp.concatenate(
            [q_ref[0, q_base:q_base + bq, g * D:(g + 1) * D]
             for g in range(G)], axis=0)
        s = jax.lax.dot_general(
            qall, kspan, (((1,), (1,)), ((), ())),
            preferred_element_type=jnp.float32,
        )
        bias = jnp.concatenate(
            [slopes_ref[h, g] * delta_masked for g in range(G)], axis=0)
        p = jnp.exp2(s + bias)
        l = jnp.sum(p, axis=1, keepdims=True)
        oall = jax.lax.dot_general(
            p, vspan, (((1,), (0,)), ((), ())),
            preferred_element_type=jnp.float32,
        ) * (1.0 / l)
        for g in range(G):
            o_ref[0, q_base:q_base + bq, g * D:(g + 1) * D] = \
                oall[g * bq:(g + 1) * bq, :]


def kernel(q, k, v):
    qh = q.reshape(B, S, H * D)
    kh = k.reshape(B, S, KVH * D)
    vh = v.reshape(B, S, KVH * D)
    slopes = jnp.asarray(
        (np.array(_slopes(H), dtype=np.float64) * LOG2E)
        .astype(np.float32).reshape(KVH, G))

    out = pl.pallas_call(
        _attn_kernel,
        grid=(B, KVH),
        in_specs=[
            pl.BlockSpec(memory_space=pltpu.SMEM),
            pl.BlockSpec((1, S, G * D), lambda b, h: (b, 0, h)),
            pl.BlockSpec((1, S, D), lambda b, h: (b, 0, h)),
            pl.BlockSpec((1, S, D), lambda b, h: (b, 0, h)),
        ],
        out_specs=pl.BlockSpec((1, S, G * D), lambda b, h: (b, 0, h)),
        out_shape=jax.ShapeDtypeStruct((B, S, H * D), jnp.float32),
        compiler_params=pltpu.CompilerParams(
            dimension_semantics=("parallel", "parallel")),
    )(slopes, qh, kh, vh)
    return out.reshape(B * S, H * D)
